# baseline (speedup 1.0000x reference)
import functools

import jax
import jax.numpy as jnp
from jax.experimental import pallas as pl
from jax.experimental.pallas import tpu as pltpu

_B, _M, _N, _D = 16, 8192, 8192, 256
_BM = 1024
_PR = 8


def _body(idx_ref, num_ref, ishape_ref, pts_ref, wt_ref, emb_ref, out_ref):
    b = pl.program_id(0)
    j = pl.program_id(1)

    hf = ishape_ref[2].astype(jnp.float32)
    wf = ishape_ref[3].astype(jnp.float32)
    kp_scale = jnp.maximum(wf, hf) * 0.7
    max_len = jnp.sqrt(wf * wf + hf * hf)
    len_scale = max_len * 0.7

    r = idx_ref[b] % _PR
    x = pts_ref[0, r, 0]
    y = pts_ref[0, r, 1]
    ln = pts_ref[0, r, 3]
    an = pts_ref[0, r, 4]

    nx = (x - wf * 0.5) / kp_scale
    ny = (y - hf * 0.5) / kp_scale
    na = (an - 45.0) / (180.0 * 0.7)
    nl = (ln - len_scale * 0.5) / len_scale

    g = (nx * wt_ref[0:1, :] + ny * wt_ref[1:2, :]
         + na * wt_ref[2:3, :] + nl * wt_ref[3:4, :])

    row = jax.lax.broadcasted_iota(jnp.int32, (_BM, 1), 0) + j * _BM
    mask = row < num_ref[b]
    out_ref[0] = emb_ref[0] + jnp.where(mask, g, 0.0)


@jax.jit
def kernel(emb, num, pts, indeces, image_shape, W):
    num = num.astype(jnp.int32)
    indeces = indeces.astype(jnp.int32)
    image_shape = image_shape.astype(jnp.int32)
    wt = W.T

    grid = (_B, _M // _BM)
    return pl.pallas_call(
        _body,
        grid_spec=pltpu.PrefetchScalarGridSpec(
            num_scalar_prefetch=3,
            grid=grid,
            in_specs=[
                pl.BlockSpec((1, _PR, 5),
                             lambda b, j, idx, n, s: (b, idx[b] // _PR, 0)),
                pl.BlockSpec((4, _D), lambda b, j, idx, n, s: (0, 0)),
                pl.BlockSpec((1, _BM, _D), lambda b, j, idx, n, s: (b, j, 0)),
            ],
            out_specs=pl.BlockSpec((1, _BM, _D),
                                   lambda b, j, idx, n, s: (b, j, 0)),
        ),
        out_shape=jax.ShapeDtypeStruct((_B, _M, _D), emb.dtype),
        compiler_params=pltpu.CompilerParams(
            dimension_semantics=("parallel", "arbitrary"),
        ),
    )(indeces, num, image_shape, pts, wt, emb)


# speedup vs baseline: 1.2893x; 1.2893x over previous
import functools

import jax
import jax.numpy as jnp
from jax.experimental import pallas as pl
from jax.experimental.pallas import tpu as pltpu

_B, _M, _N, _D = 16, 8192, 8192, 256
_BM = 2048
_PR = 8


def _body(idx_ref, num_ref, ishape_ref, pts_ref, wt_ref, emb_ref, out_ref):
    b = pl.program_id(0)
    j = pl.program_id(1)

    hf = ishape_ref[2].astype(jnp.float32)
    wf = ishape_ref[3].astype(jnp.float32)
    kp_scale = jnp.maximum(wf, hf) * 0.7
    max_len = jnp.sqrt(wf * wf + hf * hf)
    len_scale = max_len * 0.7

    r = idx_ref[b] % _PR
    x = pts_ref[0, r, 0]
    y = pts_ref[0, r, 1]
    ln = pts_ref[0, r, 3]
    an = pts_ref[0, r, 4]

    nx = (x - wf * 0.5) / kp_scale
    ny = (y - hf * 0.5) / kp_scale
    na = (an - 45.0) / (180.0 * 0.7)
    nl = (ln - len_scale * 0.5) / len_scale

    g = (nx * wt_ref[0:1, :] + ny * wt_ref[1:2, :]
         + na * wt_ref[2:3, :] + nl * wt_ref[3:4, :])

    row = jax.lax.broadcasted_iota(jnp.int32, (_BM, 1), 0) + j * _BM
    mask = row < num_ref[b]
    out_ref[0] = emb_ref[0] + jnp.where(mask, g, 0.0)


@jax.jit
def kernel(emb, num, pts, indeces, image_shape, W):
    num = num.astype(jnp.int32)
    indeces = indeces.astype(jnp.int32)
    image_shape = image_shape.astype(jnp.int32)
    wt = W.T

    grid = (_B, _M // _BM)
    return pl.pallas_call(
        _body,
        grid_spec=pltpu.PrefetchScalarGridSpec(
            num_scalar_prefetch=3,
            grid=grid,
            in_specs=[
                pl.BlockSpec((1, _PR, 5),
                             lambda b, j, idx, n, s: (b, idx[b] // _PR, 0)),
                pl.BlockSpec((4, _D), lambda b, j, idx, n, s: (0, 0)),
                pl.BlockSpec((1, _BM, _D), lambda b, j, idx, n, s: (b, j, 0)),
            ],
            out_specs=pl.BlockSpec((1, _BM, _D),
                                   lambda b, j, idx, n, s: (b, j, 0)),
        ),
        out_shape=jax.ShapeDtypeStruct((_B, _M, _D), emb.dtype),
        compiler_params=pltpu.CompilerParams(
            dimension_semantics=("parallel", "parallel"),
        ),
    )(indeces, num, image_shape, pts, wt, emb)


# speedup vs baseline: 1.4300x; 1.1091x over previous
import functools

import jax
import jax.numpy as jnp
from jax.experimental import pallas as pl
from jax.experimental.pallas import tpu as pltpu

_B, _M, _N, _D = 16, 8192, 8192, 256
_BM = 4096
_PR = 8


def _body(idx_ref, num_ref, ishape_ref, pts_ref, wt_ref, emb_ref, out_ref):
    b = pl.program_id(0)
    j = pl.program_id(1)

    hf = ishape_ref[2].astype(jnp.float32)
    wf = ishape_ref[3].astype(jnp.float32)
    kp_scale = jnp.maximum(wf, hf) * 0.7
    max_len = jnp.sqrt(wf * wf + hf * hf)
    len_scale = max_len * 0.7

    r = idx_ref[b] % _PR
    x = pts_ref[0, r, 0]
    y = pts_ref[0, r, 1]
    ln = pts_ref[0, r, 3]
    an = pts_ref[0, r, 4]

    nx = (x - wf * 0.5) / kp_scale
    ny = (y - hf * 0.5) / kp_scale
    na = (an - 45.0) / (180.0 * 0.7)
    nl = (ln - len_scale * 0.5) / len_scale

    g = (nx * wt_ref[0:1, :] + ny * wt_ref[1:2, :]
         + na * wt_ref[2:3, :] + nl * wt_ref[3:4, :])

    row = jax.lax.broadcasted_iota(jnp.int32, (_BM, 1), 0) + j * _BM
    mask = row < num_ref[b]
    out_ref[0] = emb_ref[0] + jnp.where(mask, g, 0.0)


@jax.jit
def kernel(emb, num, pts, indeces, image_shape, W):
    num = num.astype(jnp.int32)
    indeces = indeces.astype(jnp.int32)
    image_shape = image_shape.astype(jnp.int32)
    wt = W.T

    grid = (_B, _M // _BM)
    return pl.pallas_call(
        _body,
        grid_spec=pltpu.PrefetchScalarGridSpec(
            num_scalar_prefetch=3,
            grid=grid,
            in_specs=[
                pl.BlockSpec((1, _PR, 5),
                             lambda b, j, idx, n, s: (b, idx[b] // _PR, 0)),
                pl.BlockSpec((4, _D), lambda b, j, idx, n, s: (0, 0)),
                pl.BlockSpec((1, _BM, _D), lambda b, j, idx, n, s: (b, j, 0)),
            ],
            out_specs=pl.BlockSpec((1, _BM, _D),
                                   lambda b, j, idx, n, s: (b, j, 0)),
        ),
        out_shape=jax.ShapeDtypeStruct((_B, _M, _D), emb.dtype),
        compiler_params=pltpu.CompilerParams(
            dimension_semantics=("parallel", "parallel"),
        ),
    )(indeces, num, image_shape, pts, wt, emb)


# speedup vs baseline: 1.4669x; 1.0258x over previous
import functools

import jax
import jax.numpy as jnp
from jax.experimental import pallas as pl
from jax.experimental.pallas import tpu as pltpu

_B, _M, _N, _D = 16, 8192, 8192, 256
_BM = 8192
_PR = 8


def _body(idx_ref, num_ref, ishape_ref, pts_ref, wt_ref, emb_ref, out_ref):
    b = pl.program_id(0)
    j = pl.program_id(1)

    hf = ishape_ref[2].astype(jnp.float32)
    wf = ishape_ref[3].astype(jnp.float32)
    kp_scale = jnp.maximum(wf, hf) * 0.7
    max_len = jnp.sqrt(wf * wf + hf * hf)
    len_scale = max_len * 0.7

    r = idx_ref[b] % _PR
    x = pts_ref[0, r, 0]
    y = pts_ref[0, r, 1]
    ln = pts_ref[0, r, 3]
    an = pts_ref[0, r, 4]

    nx = (x - wf * 0.5) / kp_scale
    ny = (y - hf * 0.5) / kp_scale
    na = (an - 45.0) / (180.0 * 0.7)
    nl = (ln - len_scale * 0.5) / len_scale

    g = (nx * wt_ref[0:1, :] + ny * wt_ref[1:2, :]
         + na * wt_ref[2:3, :] + nl * wt_ref[3:4, :])

    row = jax.lax.broadcasted_iota(jnp.int32, (_BM, 1), 0) + j * _BM
    mask = row < num_ref[b]
    out_ref[0] = emb_ref[0] + jnp.where(mask, g, 0.0)


@jax.jit
def kernel(emb, num, pts, indeces, image_shape, W):
    num = num.astype(jnp.int32)
    indeces = indeces.astype(jnp.int32)
    image_shape = image_shape.astype(jnp.int32)
    wt = W.T

    grid = (_B, _M // _BM)
    return pl.pallas_call(
        _body,
        grid_spec=pltpu.PrefetchScalarGridSpec(
            num_scalar_prefetch=3,
            grid=grid,
            in_specs=[
                pl.BlockSpec((1, _PR, 5),
                             lambda b, j, idx, n, s: (b, idx[b] // _PR, 0)),
                pl.BlockSpec((4, _D), lambda b, j, idx, n, s: (0, 0)),
                pl.BlockSpec((1, _BM, _D), lambda b, j, idx, n, s: (b, j, 0)),
            ],
            out_specs=pl.BlockSpec((1, _BM, _D),
                                   lambda b, j, idx, n, s: (b, j, 0)),
        ),
        out_shape=jax.ShapeDtypeStruct((_B, _M, _D), emb.dtype),
        compiler_params=pltpu.CompilerParams(
            dimension_semantics=("parallel", "parallel"),
        ),
    )(indeces, num, image_shape, pts, wt, emb)
